# baseline (speedup 1.0000x reference)
import jax
import jax.numpy as jnp
from jax import lax
from jax.experimental import pallas as pl
from jax.experimental.pallas import tpu as pltpu
from jax.experimental.pallas import tpu_sc as plsc

PAD = 1
INIT = 8194
BSZ = 4
SEQ = 8192
DIM = 1024
TOKENS = BSZ * SEQ

NC = 2
NS = 16
NW = NC * NS
CHUNK = TOKENS // NW
WPR = SEQ // CHUNK
K = 32
NCHUNK = CHUNK // K
GLAG = 3
L = 16


def _body(ids_hbm, w_hbm, out_hbm, ids_v, pos_v, buf0, buf1, buf2, gsem, ssem):
    wid = lax.axis_index("s") * NC + lax.axis_index("c")
    row = wid // WPR
    sub = wid % WPR

    pltpu.sync_copy(ids_hbm.at[pl.ds(row * SEQ, SEQ)], ids_v)

    def lane_gather(x, idx):
        return lax.gather(
            x, idx[:, None],
            dimension_numbers=lax.GatherDimensionNumbers(
                offset_dims=(), collapsed_slice_dims=(0,),
                start_index_map=(0,)),
            slice_sizes=(1,),
            mode=lax.GatherScatterMode.PROMISE_IN_BOUNDS)

    iota = lax.iota(jnp.int32, L)
    last = jnp.maximum(iota, L - 1)

    def splat_last(x):
        return lane_gather(x, last)

    shift_idx = [jnp.maximum(iota - d, 0) for d in (1, 2, 4, 8)]
    shift_msk = [jnp.minimum(jnp.maximum(iota - (d - 1), 0), 1)
                 for d in (1, 2, 4, 8)]

    def csum16(x):
        for idx, mk in zip(shift_idx, shift_msk):
            x = x + lane_gather(x, idx) * mk
        return x

    def pf(i, acc):
        for u in range(4):
            v = ids_v[pl.ds((i * 4 + u) * L, L)]
            acc = acc + jnp.minimum(jnp.abs(v - PAD), 1)
        return acc

    nvec4 = sub * (CHUNK // (4 * L))
    acc = lax.fori_loop(0, nvec4, pf, jnp.zeros((L,), jnp.int32))
    carry = splat_last(csum16(acc))

    bufs = (buf0, buf1, buf2)
    nbuf = len(bufs)
    my_base = sub * CHUNK
    out_base = wid * CHUNK
    g = [None] * NCHUNK
    s = [None] * NCHUNK

    def start_scatter(j):
        g[j].wait()
        s[j] = pltpu.async_copy(
            bufs[j % nbuf], out_hbm.at[pl.ds(out_base + j * K, K)], ssem)

    for c in range(NCHUNK):
        for i in range(c * K // L, (c + 1) * K // L):
            v = ids_v[pl.ds(my_base + i * L, L)]
            m = jnp.minimum(jnp.abs(v - PAD), 1)
            cs = csum16(m) + carry
            pos_v[pl.ds(i * L, L)] = cs * m + PAD
            carry = splat_last(cs)
        if c >= 1:
            start_scatter(c - 1)
        if c >= nbuf:
            s[c - nbuf].wait()
        g[c] = pltpu.async_copy(
            w_hbm.at[pos_v.at[pl.ds(c * K, K)]], bufs[c % nbuf], gsem)
    start_scatter(NCHUNK - 1)
    for j in range(NCHUNK - nbuf, NCHUNK):
        s[j].wait()


@jax.jit
def kernel(input, weights):
    ids = input.reshape(-1).astype(jnp.int32)
    mesh = plsc.VectorSubcoreMesh(core_axis_name="c", subcore_axis_name="s")
    out = pl.kernel(
        _body,
        mesh=mesh,
        out_type=jax.ShapeDtypeStruct((TOKENS, DIM), jnp.float32),
        scratch_types=[
            pltpu.VMEM((SEQ,), jnp.int32),
            pltpu.VMEM((CHUNK,), jnp.int32),
            pltpu.VMEM((K, DIM), jnp.float32),
            pltpu.VMEM((K, DIM), jnp.float32),
            pltpu.VMEM((K, DIM), jnp.float32),
            pltpu.SemaphoreType.DMA,
            pltpu.SemaphoreType.DMA,
        ],
    )(ids, weights)
    return out.reshape(BSZ, SEQ, DIM)


# speedup vs baseline: 1.0521x; 1.0521x over previous
import jax
import jax.numpy as jnp
from jax import lax
from jax.experimental import pallas as pl
from jax.experimental.pallas import tpu as pltpu
from jax.experimental.pallas import tpu_sc as plsc

PAD = 1
INIT = 8194
BSZ = 4
SEQ = 8192
DIM = 1024
TOKENS = BSZ * SEQ

NC = 2
NS = 16
NW = NC * NS
CHUNK = TOKENS // NW
WPR = SEQ // CHUNK
K = 32
NCHUNK = CHUNK // K
GLAG = 3
L = 16


def _body(ids_hbm, w_hbm, out_hbm, ids_v, pos_v, buf0, buf1, buf2, gsem, ssem):
    wid = lax.axis_index("s") * NC + lax.axis_index("c")
    row = wid // WPR
    sub = wid % WPR

    pltpu.sync_copy(ids_hbm.at[pl.ds(row * SEQ, SEQ)], ids_v)

    def lane_gather(x, idx):
        return lax.gather(
            x, idx[:, None],
            dimension_numbers=lax.GatherDimensionNumbers(
                offset_dims=(), collapsed_slice_dims=(0,),
                start_index_map=(0,)),
            slice_sizes=(1,),
            mode=lax.GatherScatterMode.PROMISE_IN_BOUNDS)

    iota = lax.iota(jnp.int32, L)
    last = jnp.maximum(iota, L - 1)

    def splat_last(x):
        return lane_gather(x, last)

    shift_idx = [jnp.maximum(iota - d, 0) for d in (1, 2, 4, 8)]
    shift_msk = [jnp.minimum(jnp.maximum(iota - (d - 1), 0), 1)
                 for d in (1, 2, 4, 8)]

    def csum16(x):
        for idx, mk in zip(shift_idx, shift_msk):
            x = x + lane_gather(x, idx) * mk
        return x

    def pf(i, acc):
        for u in range(4):
            v = ids_v[pl.ds((i * 4 + u) * L, L)]
            acc = acc + jnp.minimum(jnp.abs(v - PAD), 1)
        return acc

    nvec4 = sub * (CHUNK // (4 * L))
    acc = lax.fori_loop(0, nvec4, pf, jnp.zeros((L,), jnp.int32))
    carry = splat_last(csum16(acc))

    bufs = (buf0, buf1, buf2)
    nbuf = len(bufs)
    my_base = sub * CHUNK
    out_base = wid * CHUNK
    g = [None] * NCHUNK
    s = [None] * NCHUNK

    def start_scatter(j):
        g[j].wait()
        s[j] = pltpu.async_copy(
            bufs[j % nbuf], out_hbm.at[pl.ds(out_base + j * K, K)], ssem)

    for c in range(NCHUNK):
        for i in range(c * K // L, (c + 1) * K // L):
            v = ids_v[pl.ds(my_base + i * L, L)]
            m = jnp.minimum(jnp.abs(v - PAD), 1)
            cs = csum16(m) + carry
            pos_v[pl.ds(i * L, L)] = cs * m + PAD
            carry = splat_last(cs)
        if c >= nbuf:
            s[c - nbuf].wait()
        g[c] = pltpu.async_copy(
            w_hbm.at[pos_v.at[pl.ds(c * K, K)]], bufs[c % nbuf], gsem)
        if c >= 1:
            start_scatter(c - 1)
    start_scatter(NCHUNK - 1)
    for j in range(NCHUNK - nbuf, NCHUNK):
        s[j].wait()


@jax.jit
def kernel(input, weights):
    ids = input.reshape(-1).astype(jnp.int32)
    mesh = plsc.VectorSubcoreMesh(core_axis_name="c", subcore_axis_name="s")
    out = pl.kernel(
        _body,
        mesh=mesh,
        out_type=jax.ShapeDtypeStruct((TOKENS, DIM), jnp.float32),
        scratch_types=[
            pltpu.VMEM((SEQ,), jnp.int32),
            pltpu.VMEM((CHUNK,), jnp.int32),
            pltpu.VMEM((K, DIM), jnp.float32),
            pltpu.VMEM((K, DIM), jnp.float32),
            pltpu.VMEM((K, DIM), jnp.float32),
            pltpu.SemaphoreType.DMA,
            pltpu.SemaphoreType.DMA,
        ],
    )(ids, weights)
    return out.reshape(BSZ, SEQ, DIM)


# speedup vs baseline: 1.0526x; 1.0005x over previous
import jax
import jax.numpy as jnp
from jax import lax
from jax.experimental import pallas as pl
from jax.experimental.pallas import tpu as pltpu
from jax.experimental.pallas import tpu_sc as plsc

PAD = 1
BSZ = 4
SEQ = 8192
DIM = 1024
TOKENS = BSZ * SEQ

NC = 2
NS = 16
NW = NC * NS
CHUNK = TOKENS // NW
WPR = SEQ // CHUNK
K = 32
NCHUNK = CHUNK // K
L = 16


def _body(ids_hbm, w_hbm, out_hbm, ids_v, pos_v, buf0, buf1, buf2, gsem, ssem):
    wid = lax.axis_index("s") * NC + lax.axis_index("c")
    row = wid // WPR
    sub = wid % WPR

    pltpu.sync_copy(ids_hbm.at[pl.ds(row * SEQ, SEQ)], ids_v)

    def lane_gather(x, idx):
        return lax.gather(
            x, idx[:, None],
            dimension_numbers=lax.GatherDimensionNumbers(
                offset_dims=(), collapsed_slice_dims=(0,),
                start_index_map=(0,)),
            slice_sizes=(1,),
            mode=lax.GatherScatterMode.PROMISE_IN_BOUNDS)

    iota = lax.iota(jnp.int32, L)
    last = jnp.maximum(iota, L - 1)

    def splat_last(x):
        return lane_gather(x, last)

    shift_idx = [jnp.maximum(iota - d, 0) for d in (1, 2, 4, 8)]
    shift_msk = [jnp.minimum(jnp.maximum(iota - (d - 1), 0), 1)
                 for d in (1, 2, 4, 8)]

    def csum16(x):
        for idx, mk in zip(shift_idx, shift_msk):
            x = x + lane_gather(x, idx) * mk
        return x

    def pf(i, acc):
        for u in range(4):
            v = ids_v[pl.ds((i * 4 + u) * L, L)]
            acc = acc + jnp.minimum(jnp.abs(v - PAD), 1)
        return acc

    nvec4 = sub * (CHUNK // (4 * L))
    acc = lax.fori_loop(0, nvec4, pf, jnp.zeros((L,), jnp.int32))
    carry = splat_last(csum16(acc))

    bufs = (buf0, buf1, buf2)
    nbuf = len(bufs)
    my_base = sub * CHUNK
    out_base = wid * CHUNK
    g = [None] * NCHUNK
    s = [None] * NCHUNK

    def start_scatter(j):
        g[j].wait()
        s[j] = pltpu.async_copy(
            bufs[j % nbuf], out_hbm.at[pl.ds(out_base + j * K, K)], ssem)

    for c in range(NCHUNK):
        for i in range(c * K // L, (c + 1) * K // L):
            v = ids_v[pl.ds(my_base + i * L, L)]
            m = jnp.minimum(jnp.abs(v - PAD), 1)
            cs = csum16(m) + carry
            pos_v[pl.ds(i * L, L)] = cs * m + PAD
            carry = splat_last(cs)
        if c >= nbuf:
            s[c - nbuf].wait()
        g[c] = pltpu.async_copy(
            w_hbm.at[pos_v.at[pl.ds(c * K, K)]], bufs[c % nbuf], gsem)
        if c >= 1:
            start_scatter(c - 1)
    start_scatter(NCHUNK - 1)
    for j in range(NCHUNK - nbuf, NCHUNK):
        s[j].wait()


@jax.jit
def kernel(input, weights):
    ids = input.reshape(-1).astype(jnp.int32)
    mesh = plsc.VectorSubcoreMesh(core_axis_name="c", subcore_axis_name="s")
    out = pl.kernel(
        _body,
        mesh=mesh,
        out_type=jax.ShapeDtypeStruct((TOKENS, DIM), jnp.float32),
        scratch_types=[
            pltpu.VMEM((SEQ,), jnp.int32),
            pltpu.VMEM((CHUNK,), jnp.int32),
            pltpu.VMEM((K, DIM), jnp.float32),
            pltpu.VMEM((K, DIM), jnp.float32),
            pltpu.VMEM((K, DIM), jnp.float32),
            pltpu.SemaphoreType.DMA,
            pltpu.SemaphoreType.DMA,
        ],
    )(ids, weights)
    return out.reshape(BSZ, SEQ, DIM)
